# baseline (speedup 1.0000x reference)
import functools

import jax
import jax.numpy as jnp
from jax import lax
from jax.experimental import pallas as pl
from jax.experimental.pallas import tpu as pltpu
from jax.experimental.pallas import tpu_sc as plsc

NC = 2
NS = 16
NW = NC * NS
LANES = 16
GSUB = 128


def _sc_embed(idx_flat, tok_flat, pos_flat, B, T, V, D):
    b_per_w = B // NW
    nbblk = b_per_w // 128
    ngath = b_per_w // GSUB
    groups16 = b_per_w // LANES
    tv = T * V
    rows_per_tile = tv // NS
    t_per_tile = rows_per_tile // V
    mesh = plsc.VectorSubcoreMesh(
        core_axis_name="c", subcore_axis_name="s", num_cores=NC, num_subcores=NS
    )

    @functools.partial(
        pl.kernel,
        out_type=(
            jax.ShapeDtypeStruct((T, D // 8, B // 128, 8, 128), jnp.float32),
            jax.ShapeDtypeStruct((NC * tv, D), jnp.float32),
        ),
        mesh=mesh,
        compiler_params=pltpu.CompilerParams(
            needs_layout_passes=False, use_tc_tiling_on_sc=False
        ),
        scratch_types=[
            pltpu.VMEM((V * D,), jnp.float32),
            pltpu.VMEM((T * D,), jnp.float32),
            pltpu.VMEM((rows_per_tile, D), jnp.float32),
            pltpu.VMEM((B // NW * T,), jnp.int32),
            pltpu.VMEM((B // NW,), jnp.int32),
            pltpu.VMEM((B // NW, D), jnp.float32),
            pltpu.VMEM((D // 8, B // NW // 128, 8, 128), jnp.float32),
            pltpu.SemaphoreType.DMA,
            pltpu.SemaphoreType.DMA,
        ],
    )
    def k(idx_hbm, tok_hbm, pos_hbm, out_hbm, fused_hbm,
          tok_v, pos_v, build_v, idxb_v, fidx_v, rows_v, trans_v, sem, sem_o):
        c = lax.axis_index("c")
        s = lax.axis_index("s")
        wid = s * NC + c
        pltpu.sync_copy(tok_hbm, tok_v)
        pltpu.sync_copy(pos_hbm, pos_v)
        iota = lax.iota(jnp.int32, LANES)
        iotaT = iota * T
        ihig = iota // 8
        ilow = iota % 8

        for dd in range(D // LANES):
            pos_chunks = [
                pos_v[pl.ds((s * t_per_tile + tt) * D + dd * LANES, LANES)]
                for tt in range(t_per_tile)
            ]
            for v in range(V):
                tokc = tok_v[pl.ds(v * D + dd * LANES, LANES)]
                for tt in range(t_per_tile):
                    build_v[tt * V + v, pl.ds(dd * LANES, LANES)] = (
                        tokc + pos_chunks[tt]
                    )
        pltpu.sync_copy(
            build_v,
            fused_hbm.at[pl.ds(c * tv + s * rows_per_tile, rows_per_tile)],
        )
        plsc.subcore_barrier()

        pltpu.sync_copy(idx_hbm.at[pl.ds(wid * b_per_w * T, b_per_w * T)],
                        idxb_v)
        cbase = c * tv

        def t_body(t, carry):
            tbase = t * V + cbase
            for g in range(groups16):
                vidx = plsc.load_gather(
                    idxb_v, [iotaT + (g * LANES * T + t)]
                )
                fidx_v[pl.ds(g * LANES, LANES)] = vidx + tbase
            copies = [
                pltpu.async_copy(
                    fused_hbm.at[fidx_v.at[pl.ds(j * GSUB, GSUB)]],
                    rows_v.at[pl.ds(j * GSUB, GSUB)],
                    sem,
                )
                for j in range(ngath)
            ]

            @pl.when(t > 0)
            def _():
                for dblk in range(D // 8):
                    pltpu.make_async_copy(
                        trans_v.at[dblk],
                        out_hbm.at[t, dblk, pl.ds(wid * nbblk, nbblk)],
                        sem_o,
                    ).wait()

            for cp in copies:
                cp.wait()

            @plsc.parallel_loop(0, nbblk)
            def transpose_body(bb):
                rowvecs = [
                    jnp.full((LANES,), bb * 128 + g * LANES, jnp.int32) + iota
                    for g in range(128 // LANES)
                ]
                for d in range(D):
                    dcol = jnp.full((LANES,), d, jnp.int32)
                    for g in range(128 // LANES):
                        rvec = plsc.load_gather(rows_v, [rowvecs[g], dcol])
                        trans_v[d // 8, bb, d % 8,
                                pl.ds(g * LANES, LANES)] = rvec

            for dblk in range(D // 8):
                pltpu.async_copy(
                    trans_v.at[dblk],
                    out_hbm.at[t, dblk, pl.ds(wid * nbblk, nbblk)],
                    sem_o,
                )
            return carry

        lax.fori_loop(0, T, t_body, 0)
        for dblk in range(D // 8):
            pltpu.make_async_copy(
                trans_v.at[dblk],
                out_hbm.at[T - 1, dblk, pl.ds(wid * nbblk, nbblk)],
                sem_o,
            ).wait()

    out5d, _ = k(idx_flat, tok_flat, pos_flat)
    return out5d


def kernel(idx, tok_emb, pos_emb):
    B, T = idx.shape
    V, D = tok_emb.shape
    out5d = _sc_embed(
        idx.reshape(-1),
        tok_emb.reshape(-1),
        pos_emb.reshape(-1),
        B, T, V, D,
    )
    return out5d.transpose(2, 4, 0, 1, 3).reshape(B, T, D)


# speedup vs baseline: 1.6422x; 1.6422x over previous
import functools

import jax
import jax.numpy as jnp
from jax import lax
from jax.experimental import pallas as pl
from jax.experimental.pallas import tpu as pltpu
from jax.experimental.pallas import tpu_sc as plsc

NC = 2
NS = 16
NW = NC * NS
LANES = 16
GSUB = 128


def _sc_embed(idx_flat, tok_flat, pos_flat, B, T, V, D):
    b_per_w = B // NW
    nbblk = b_per_w // 128
    ngath = b_per_w // GSUB
    groups16 = b_per_w // LANES
    tv = T * V
    rows_per_tile = tv // NS
    t_per_tile = rows_per_tile // V
    mesh = plsc.VectorSubcoreMesh(
        core_axis_name="c", subcore_axis_name="s", num_cores=NC, num_subcores=NS
    )

    @functools.partial(
        pl.kernel,
        out_type=(
            jax.ShapeDtypeStruct((T, D // 8, B // 128, 8, 128), jnp.float32),
            jax.ShapeDtypeStruct((NC * tv, D), jnp.float32),
        ),
        mesh=mesh,
        compiler_params=pltpu.CompilerParams(
            needs_layout_passes=False, use_tc_tiling_on_sc=False
        ),
        scratch_types=[
            pltpu.VMEM((V * D,), jnp.float32),
            pltpu.VMEM((T * D,), jnp.float32),
            pltpu.VMEM((rows_per_tile, D), jnp.float32),
            pltpu.VMEM((B // NW * T,), jnp.int32),
            pltpu.VMEM((B // NW,), jnp.int32),
            pltpu.VMEM((B // NW, D), jnp.float32),
            pltpu.VMEM((D // 8, B // NW // 128, 8, 128), jnp.float32),
            pltpu.SemaphoreType.DMA,
            pltpu.SemaphoreType.DMA,
        ],
    )
    def k(idx_hbm, tok_hbm, pos_hbm, out_hbm, fused_hbm,
          tok_v, pos_v, build_v, idxb_v, fidx_v, rows_v, trans_v, sem, sem_o):
        c = lax.axis_index("c")
        s = lax.axis_index("s")
        wid = s * NC + c
        pltpu.sync_copy(tok_hbm, tok_v)
        pltpu.sync_copy(pos_hbm, pos_v)
        iota = lax.iota(jnp.int32, LANES)
        iotaT = iota * T
        ihig = iota // 8
        ilow = iota % 8

        for dd in range(D // LANES):
            pos_chunks = [
                pos_v[pl.ds((s * t_per_tile + tt) * D + dd * LANES, LANES)]
                for tt in range(t_per_tile)
            ]
            for v in range(V):
                tokc = tok_v[pl.ds(v * D + dd * LANES, LANES)]
                for tt in range(t_per_tile):
                    build_v[tt * V + v, pl.ds(dd * LANES, LANES)] = (
                        tokc + pos_chunks[tt]
                    )
        pltpu.sync_copy(
            build_v,
            fused_hbm.at[pl.ds(c * tv + s * rows_per_tile, rows_per_tile)],
        )
        plsc.subcore_barrier()

        pltpu.sync_copy(idx_hbm.at[pl.ds(wid * b_per_w * T, b_per_w * T)],
                        idxb_v)
        cbase = c * tv

        def t_body(t, carry):
            tbase = t * V + cbase
            for g in range(groups16):
                vidx = plsc.load_gather(
                    idxb_v, [iotaT + (g * LANES * T + t)]
                )
                fidx_v[pl.ds(g * LANES, LANES)] = vidx + tbase
            copies = [
                pltpu.async_copy(
                    fused_hbm.at[fidx_v.at[pl.ds(j * GSUB, GSUB)]],
                    rows_v.at[pl.ds(j * GSUB, GSUB)],
                    sem,
                )
                for j in range(ngath)
            ]

            @pl.when(t > 0)
            def _():
                for dblk in range(D // 8):
                    pltpu.make_async_copy(
                        trans_v.at[dblk],
                        out_hbm.at[t, dblk, pl.ds(wid * nbblk, nbblk)],
                        sem_o,
                    ).wait()

            for cp in copies:
                cp.wait()

            @plsc.parallel_loop(0, D, unroll=4)
            def transpose_body(d):
                dcol = jnp.full((LANES,), d, jnp.int32)
                for bb in range(nbblk):
                    for g in range(128 // LANES):
                        rvec = plsc.load_gather(
                            rows_v,
                            [jnp.full((LANES,), bb * 128 + g * LANES,
                                      jnp.int32) + iota, dcol],
                        )
                        trans_v[d // 8, bb, d % 8,
                                pl.ds(g * LANES, LANES)] = rvec

            for dblk in range(D // 8):
                pltpu.async_copy(
                    trans_v.at[dblk],
                    out_hbm.at[t, dblk, pl.ds(wid * nbblk, nbblk)],
                    sem_o,
                )
            return carry

        lax.fori_loop(0, T, t_body, 0)
        for dblk in range(D // 8):
            pltpu.make_async_copy(
                trans_v.at[dblk],
                out_hbm.at[T - 1, dblk, pl.ds(wid * nbblk, nbblk)],
                sem_o,
            ).wait()

    out5d, _ = k(idx_flat, tok_flat, pos_flat)
    return out5d


def kernel(idx, tok_emb, pos_emb):
    B, T = idx.shape
    V, D = tok_emb.shape
    out5d = _sc_embed(
        idx.reshape(-1),
        tok_emb.reshape(-1),
        pos_emb.reshape(-1),
        B, T, V, D,
    )
    return out5d.transpose(2, 4, 0, 1, 3).reshape(B, T, D)


# speedup vs baseline: 2.0111x; 1.2246x over previous
import functools

import jax
import jax.numpy as jnp
from jax import lax
from jax.experimental import pallas as pl
from jax.experimental.pallas import tpu as pltpu
from jax.experimental.pallas import tpu_sc as plsc

NC = 2
NS = 16
NW = NC * NS
LANES = 16

CHUNK_TOK = 512
GSUB = 128


def _sc_embed(idx_flat, tok_flat, pos_flat, B, T, V, D):
    ntok = B * T
    tok_per_w = ntok // NW
    n_chunks = tok_per_w // CHUNK_TOK
    ngath = CHUNK_TOK // GSUB
    groups16 = CHUNK_TOK // LANES
    tv = T * V
    rows_per_tile = tv // NS
    t_per_tile = rows_per_tile // V
    mesh = plsc.VectorSubcoreMesh(
        core_axis_name="c", subcore_axis_name="s", num_cores=NC, num_subcores=NS
    )

    @functools.partial(
        pl.kernel,
        out_type=(
            jax.ShapeDtypeStruct((ntok, D), jnp.float32),
            jax.ShapeDtypeStruct((NC * tv, D), jnp.float32),
        ),
        mesh=mesh,
        compiler_params=pltpu.CompilerParams(
            needs_layout_passes=False, use_tc_tiling_on_sc=False
        ),
        scratch_types=[
            pltpu.VMEM((V * D,), jnp.float32),
            pltpu.VMEM((T * D,), jnp.float32),
            pltpu.VMEM((rows_per_tile, D), jnp.float32),
            pltpu.VMEM((CHUNK_TOK,), jnp.int32),
            pltpu.VMEM((CHUNK_TOK,), jnp.int32),
            pltpu.VMEM((CHUNK_TOK,), jnp.int32),
            pltpu.VMEM((CHUNK_TOK, D), jnp.float32),
            pltpu.SemaphoreType.DMA,
        ],
    )
    def k(idx_hbm, tok_hbm, pos_hbm, out_hbm, fused_hbm,
          tok_v, pos_v, build_v, idx_v, tbase_v, fidx_v, rows_v, sem):
        c = lax.axis_index("c")
        s = lax.axis_index("s")
        wid = s * NC + c
        pltpu.sync_copy(tok_hbm, tok_v)
        pltpu.sync_copy(pos_hbm, pos_v)
        iota = lax.iota(jnp.int32, LANES)

        for dd in range(D // LANES):
            pos_chunks = [
                pos_v[pl.ds((s * t_per_tile + tt) * D + dd * LANES, LANES)]
                for tt in range(t_per_tile)
            ]
            for v in range(V):
                tokc = tok_v[pl.ds(v * D + dd * LANES, LANES)]
                for tt in range(t_per_tile):
                    build_v[tt * V + v, pl.ds(dd * LANES, LANES)] = (
                        tokc + pos_chunks[tt]
                    )
        pltpu.sync_copy(
            build_v,
            fused_hbm.at[pl.ds(c * tv + s * rows_per_tile, rows_per_tile)],
        )
        plsc.subcore_barrier()

        cbase = c * tv
        for g in range(groups16):
            toff = (g % (T // LANES)) * LANES
            tbase_v[pl.ds(g * LANES, LANES)] = (iota + toff) * V + cbase

        tok0_w = wid * tok_per_w

        def chunk_body(ch, carry):
            tok0 = tok0_w + ch * CHUNK_TOK
            pltpu.sync_copy(idx_hbm.at[pl.ds(tok0, CHUNK_TOK)], idx_v)
            for g in range(groups16):
                fidx_v[pl.ds(g * LANES, LANES)] = (
                    idx_v[pl.ds(g * LANES, LANES)]
                    + tbase_v[pl.ds(g * LANES, LANES)]
                )
            copies = [
                pltpu.async_copy(
                    fused_hbm.at[fidx_v.at[pl.ds(j * GSUB, GSUB)]],
                    rows_v.at[pl.ds(j * GSUB, GSUB)],
                    sem,
                )
                for j in range(ngath)
            ]
            for cp in copies:
                cp.wait()
            pltpu.sync_copy(rows_v, out_hbm.at[pl.ds(tok0, CHUNK_TOK)])
            return carry

        lax.fori_loop(0, n_chunks, chunk_body, 0)

    out2d, _ = k(idx_flat, tok_flat, pos_flat)
    return out2d


def kernel(idx, tok_emb, pos_emb):
    B, T = idx.shape
    V, D = tok_emb.shape
    out2d = _sc_embed(
        idx.reshape(-1),
        tok_emb.reshape(-1),
        pos_emb.reshape(-1),
        B, T, V, D,
    )
    return out2d.reshape(B, T, D)


# speedup vs baseline: 3.1134x; 1.5481x over previous
import functools

import jax
import jax.numpy as jnp
from jax import lax
from jax.experimental import pallas as pl
from jax.experimental.pallas import tpu as pltpu
from jax.experimental.pallas import tpu_sc as plsc

NC = 2
NS = 16
NW = NC * NS
LANES = 16
GSUB = 128


def _sc_embed(idx_flat, tok_flat, pos_flat, B, T, V, D):
    b_per_w = B // NW
    nbblk = b_per_w // 128
    ngath = b_per_w // GSUB
    groups16 = b_per_w // LANES
    tv = T * V
    rows_per_tile = tv // NS
    t_per_tile = rows_per_tile // V
    mesh = plsc.VectorSubcoreMesh(
        core_axis_name="c", subcore_axis_name="s", num_cores=NC, num_subcores=NS
    )

    @functools.partial(
        pl.kernel,
        out_type=(
            jax.ShapeDtypeStruct((T, D // 8, B // 128, 8, 128), jnp.float32),
            jax.ShapeDtypeStruct((NC * tv, D), jnp.float32),
        ),
        mesh=mesh,
        compiler_params=pltpu.CompilerParams(
            needs_layout_passes=False, use_tc_tiling_on_sc=False
        ),
        scratch_types=[
            pltpu.VMEM((V * D,), jnp.float32),
            pltpu.VMEM((T * V // NS // V * D,), jnp.float32),
            pltpu.VMEM((128, D + 1), jnp.float32),
            pltpu.VMEM((rows_per_tile, D), jnp.float32),
            pltpu.VMEM((B // NW * T,), jnp.int32),
            pltpu.VMEM((B // NW,), jnp.int32),
            pltpu.VMEM((B // NW, D), jnp.float32),
            pltpu.VMEM((D // 8, B // NW // 128, 8, 128), jnp.float32),
            pltpu.SemaphoreType.DMA,
            pltpu.SemaphoreType.DMA,
        ],
    )
    def k(idx_hbm, tok_hbm, pos_hbm, out_hbm, fused_hbm,
          tok_v, pos_v, rows65_v, build_v, idxb_v, fidx_v, rows_v, trans_v,
          sem, sem_o):
        c = lax.axis_index("c")
        s = lax.axis_index("s")
        wid = s * NC + c
        pltpu.sync_copy(tok_hbm, tok_v)
        pltpu.sync_copy(
            pos_hbm.at[pl.ds(s * t_per_tile * D, t_per_tile * D)], pos_v
        )
        iota = lax.iota(jnp.int32, LANES)
        iotaT = iota * T
        ihig = iota // 8
        ilow = iota % 8

        for dd in range(D // LANES):
            pos_chunks = [
                pos_v[pl.ds(tt * D + dd * LANES, LANES)]
                for tt in range(t_per_tile)
            ]
            for v in range(V):
                tokc = tok_v[pl.ds(v * D + dd * LANES, LANES)]
                for tt in range(t_per_tile):
                    build_v[tt * V + v, pl.ds(dd * LANES, LANES)] = (
                        tokc + pos_chunks[tt]
                    )
        pltpu.sync_copy(
            build_v,
            fused_hbm.at[pl.ds(c * tv + s * rows_per_tile, rows_per_tile)],
        )
        plsc.subcore_barrier()

        pltpu.sync_copy(idx_hbm.at[pl.ds(wid * b_per_w * T, b_per_w * T)],
                        idxb_v)
        cbase = c * tv

        def t_body(t, carry):
            tbase = t * V + cbase
            for g in range(groups16):
                vidx = plsc.load_gather(
                    idxb_v, [iotaT + (g * LANES * T + t)]
                )
                fidx_v[pl.ds(g * LANES, LANES)] = vidx + tbase
            copies = [
                pltpu.async_copy(
                    fused_hbm.at[fidx_v.at[pl.ds(j * GSUB, GSUB)]],
                    rows_v.at[pl.ds(j * GSUB, GSUB)],
                    sem,
                )
                for j in range(ngath)
            ]

            @pl.when(t > 0)
            def _():
                for dblk in range(D // 8):
                    pltpu.make_async_copy(
                        trans_v.at[dblk],
                        out_hbm.at[t, dblk, pl.ds(wid * nbblk, nbblk)],
                        sem_o,
                    ).wait()

            for cp in copies:
                cp.wait()

            for bb in range(nbblk):
                @plsc.parallel_loop(0, 128)
                def hop1(bi):
                    for dc in range(D // LANES):
                        rows65_v[bi, pl.ds(dc * LANES, LANES)] = (
                            rows_v[bb * 128 + bi, pl.ds(dc * LANES, LANES)]
                        )

                @plsc.parallel_loop(0, D)
                def hop2(d):
                    dcol = jnp.full((LANES,), d, jnp.int32)
                    for g in range(128 // LANES):
                        rvec = plsc.load_gather(
                            rows65_v,
                            [jnp.full((LANES,), g * LANES, jnp.int32) + iota,
                             dcol],
                        )
                        trans_v[d // 8, bb, d % 8,
                                pl.ds(g * LANES, LANES)] = rvec

            for dblk in range(D // 8):
                pltpu.async_copy(
                    trans_v.at[dblk],
                    out_hbm.at[t, dblk, pl.ds(wid * nbblk, nbblk)],
                    sem_o,
                )
            return carry

        lax.fori_loop(0, T, t_body, 0)
        for dblk in range(D // 8):
            pltpu.make_async_copy(
                trans_v.at[dblk],
                out_hbm.at[T - 1, dblk, pl.ds(wid * nbblk, nbblk)],
                sem_o,
            ).wait()

    out5d, _ = k(idx_flat, tok_flat, pos_flat)
    return out5d


def kernel(idx, tok_emb, pos_emb):
    B, T = idx.shape
    V, D = tok_emb.shape
    out5d = _sc_embed(
        idx.reshape(-1),
        tok_emb.reshape(-1),
        pos_emb.reshape(-1),
        B, T, V, D,
    )
    return out5d.transpose(2, 4, 0, 1, 3).reshape(B, T, D)


# speedup vs baseline: 3.5808x; 1.1501x over previous
import functools

import jax
import jax.numpy as jnp
from jax import lax
from jax.experimental import pallas as pl
from jax.experimental.pallas import tpu as pltpu
from jax.experimental.pallas import tpu_sc as plsc

NC = 2
NS = 16
NW = NC * NS
LANES = 16
GSUB = 128


def _sc_embed(idx_flat, tok_flat, pos_flat, B, T, V, D):
    b_per_w = B // NW
    nbblk = b_per_w // 128
    ngath = b_per_w // GSUB
    groups16 = b_per_w // LANES
    tv = T * V
    rows_per_tile = tv // NS
    t_per_tile = rows_per_tile // V
    mesh = plsc.VectorSubcoreMesh(
        core_axis_name="c", subcore_axis_name="s", num_cores=NC, num_subcores=NS
    )

    @functools.partial(
        pl.kernel,
        out_type=(
            jax.ShapeDtypeStruct((T, D // 8, B // 128, 8, 128), jnp.float32),
            jax.ShapeDtypeStruct((NC * tv, D), jnp.float32),
        ),
        mesh=mesh,
        compiler_params=pltpu.CompilerParams(
            needs_layout_passes=False, use_tc_tiling_on_sc=False
        ),
        scratch_types=[
            pltpu.VMEM((V * D,), jnp.float32),
            pltpu.VMEM((T * V // NS // V * D,), jnp.float32),
            pltpu.VMEM((128, D + 1), jnp.float32),
            pltpu.VMEM((rows_per_tile, D), jnp.float32),
            pltpu.VMEM((B // NW * T,), jnp.int32),
            pltpu.VMEM((B // NW,), jnp.int32),
            pltpu.VMEM((B // NW, D), jnp.float32),
            pltpu.VMEM((D // 8, B // NW // 128, 8, 128), jnp.float32),
            pltpu.SemaphoreType.DMA,
            pltpu.SemaphoreType.DMA,
        ],
    )
    def k(idx_hbm, tok_hbm, pos_hbm, out_hbm, fused_hbm,
          tok_v, pos_v, rows65_v, build_v, idxb_v, fidx_v, rows_v, trans_v,
          sem, sem_o):
        c = lax.axis_index("c")
        s = lax.axis_index("s")
        wid = s * NC + c
        pltpu.sync_copy(tok_hbm, tok_v)
        pltpu.sync_copy(
            pos_hbm.at[pl.ds(s * t_per_tile * D, t_per_tile * D)], pos_v
        )
        iota = lax.iota(jnp.int32, LANES)
        iotaT = iota * T
        ihig = iota // 8
        ilow = iota % 8

        for dd in range(D // LANES):
            pos_chunks = [
                pos_v[pl.ds(tt * D + dd * LANES, LANES)]
                for tt in range(t_per_tile)
            ]
            for v in range(V):
                tokc = tok_v[pl.ds(v * D + dd * LANES, LANES)]
                for tt in range(t_per_tile):
                    build_v[tt * V + v, pl.ds(dd * LANES, LANES)] = (
                        tokc + pos_chunks[tt]
                    )
        pltpu.sync_copy(
            build_v,
            fused_hbm.at[pl.ds(c * tv + s * rows_per_tile, rows_per_tile)],
        )
        plsc.subcore_barrier()

        pltpu.sync_copy(idx_hbm.at[pl.ds(wid * b_per_w * T, b_per_w * T)],
                        idxb_v)
        cbase = c * tv

        def t_body(t, carry):
            tbase = t * V + cbase
            for g in range(groups16):
                vidx = plsc.load_gather(
                    idxb_v, [iotaT + (g * LANES * T + t)]
                )
                fidx_v[pl.ds(g * LANES, LANES)] = vidx + tbase
            copies = [
                pltpu.async_copy(
                    fused_hbm.at[fidx_v.at[pl.ds(j * GSUB, GSUB)]],
                    rows_v.at[pl.ds(j * GSUB, GSUB)],
                    sem,
                )
                for j in range(ngath)
            ]

            @pl.when(t > 0)
            def _():
                for dblk in range(D // 8):
                    pltpu.make_async_copy(
                        trans_v.at[dblk],
                        out_hbm.at[t, dblk, pl.ds(wid * nbblk, nbblk)],
                        sem_o,
                    ).wait()

            for bb in range(nbblk):
                copies[bb].wait()
                @plsc.parallel_loop(0, 128)
                def hop1(bi):
                    for dc in range(D // LANES):
                        rows65_v[bi, pl.ds(dc * LANES, LANES)] = (
                            rows_v[bb * 128 + bi, pl.ds(dc * LANES, LANES)]
                        )

                @plsc.parallel_loop(0, D)
                def hop2(d):
                    dcol = jnp.full((LANES,), d, jnp.int32)
                    for g in range(128 // LANES):
                        rvec = plsc.load_gather(
                            rows65_v,
                            [jnp.full((LANES,), g * LANES, jnp.int32) + iota,
                             dcol],
                        )
                        trans_v[d // 8, bb, d % 8,
                                pl.ds(g * LANES, LANES)] = rvec

            for dblk in range(D // 8):
                pltpu.async_copy(
                    trans_v.at[dblk],
                    out_hbm.at[t, dblk, pl.ds(wid * nbblk, nbblk)],
                    sem_o,
                )
            return carry

        lax.fori_loop(0, T, t_body, 0)
        for dblk in range(D // 8):
            pltpu.make_async_copy(
                trans_v.at[dblk],
                out_hbm.at[T - 1, dblk, pl.ds(wid * nbblk, nbblk)],
                sem_o,
            ).wait()

    out5d, _ = k(idx_flat, tok_flat, pos_flat)
    return out5d


def kernel(idx, tok_emb, pos_emb):
    B, T = idx.shape
    V, D = tok_emb.shape
    out5d = _sc_embed(
        idx.reshape(-1),
        tok_emb.reshape(-1),
        pos_emb.reshape(-1),
        B, T, V, D,
    )
    return out5d.transpose(2, 4, 0, 1, 3).reshape(B, T, D)
